# baseline (speedup 1.0000x reference)
import functools

import jax
import jax.numpy as jnp
from jax import lax
from jax.experimental import pallas as pl
from jax.experimental.pallas import tpu as pltpu
from jax.experimental.pallas import tpu_sc as plsc

VOCAB = 1000000
DIM = 32
BATCH = 16384
SEQ = 50
TOTAL = BATCH * SEQ
NW = 32
BPW = BATCH // NW
NBLK = BPW // 128

_mesh = plsc.VectorSubcoreMesh(core_axis_name="c", subcore_axis_name="s")


@functools.partial(
    pl.kernel,
    mesh=_mesh,
    compiler_params=pltpu.CompilerParams(use_tc_tiling_on_sc=False,
                                         needs_layout_passes=False),
    out_type=jax.ShapeDtypeStruct((SEQ, DIM, BATCH), jnp.float32),
    scratch_types=[
        pltpu.VMEM((BPW,), jnp.int32),
        pltpu.VMEM((BPW,), jnp.int32),
        pltpu.VMEM((BPW, DIM), jnp.float32),
        pltpu.VMEM((BPW, DIM), jnp.float32),
        pltpu.VMEM((DIM * BPW,), jnp.float32),
        pltpu.VMEM((DIM * BPW,), jnp.float32),
        pltpu.SemaphoreType.DMA,
        pltpu.SemaphoreType.DMA,
        pltpu.SemaphoreType.DMA,
        pltpu.SemaphoreType.DMA,
        pltpu.SemaphoreType.DMA,
        pltpu.SemaphoreType.DMA,
    ],
)
def _lookup_kernel(idx_hbm, table_hbm, out_hbm, idx_a, idx_b, rows_a,
                   rows_b, tr_a, tr_b, sia, sib, sga, sgb, swa, swb):
    wid = lax.axis_index("s") * 2 + lax.axis_index("c")
    b0 = wid * BPW

    def idx_load(s, buf, sem):
        return pltpu.async_copy(idx_hbm.at[pl.ds(s * BATCH + b0, BPW)],
                                buf, sem)

    def gather(ibuf, rbuf, sem):
        return pltpu.async_copy(table_hbm.at[ibuf], rbuf, sem)

    lane = lax.iota(jnp.int32, 16)

    l0 = lane * BPW
    l1 = l0 + 16 * BPW

    def drain_writes(s, tbuf, swx):
        def wbody(d, carry):
            pltpu.make_async_copy(
                tbuf.at[pl.ds(d * BPW, BPW)],
                out_hbm.at[s, d, pl.ds(b0, BPW)], swx).wait()
            return carry
        lax.fori_loop(0, DIM, wbody, 0)

    def transpose_write(t, s, rbuf, tbuf, swx):
        @pl.when(t > 0)
        def _():
            drain_writes(s, tbuf, swx)

        def rbody(r, carry):
            base = r * 8
            for j in range(8):
                bl = base + j
                v0 = rbuf[bl, pl.ds(0, 16)]
                v1 = rbuf[bl, pl.ds(16, 16)]
                plsc.store_scatter(tbuf, [l0 + bl], v0)
                plsc.store_scatter(tbuf, [l1 + bl], v1)
            return carry

        lax.fori_loop(0, BPW // 8, rbody, 0)

        def dbody(d, carry):
            pltpu.async_copy(tbuf.at[pl.ds(d * BPW, BPW)],
                             out_hbm.at[s, d, pl.ds(b0, BPW)], swx)
            return carry
        lax.fori_loop(0, DIM, dbody, 0)

    idx_load(0, idx_a, sia).wait()
    idx_load(1, idx_b, sib)
    gather(idx_a, rows_a, sga)

    def body(t, carry):
        pltpu.make_async_copy(table_hbm.at[idx_a], rows_a, sga).wait()
        pltpu.make_async_copy(
            idx_hbm.at[pl.ds(b0, BPW)], idx_b, sib).wait()
        gather(idx_b, rows_b, sgb)

        @pl.when(t < SEQ // 2 - 1)
        def _():
            idx_load(2 * t + 2, idx_a, sia)

        transpose_write(t, 2 * t, rows_a, tr_a, swa)
        pltpu.make_async_copy(table_hbm.at[idx_b], rows_b, sgb).wait()

        @pl.when(t < SEQ // 2 - 1)
        def _():
            pltpu.make_async_copy(
                idx_hbm.at[pl.ds(b0, BPW)], idx_a, sia).wait()
            gather(idx_a, rows_a, sga)
            idx_load(2 * t + 3, idx_b, sib)

        transpose_write(t, 2 * t + 1, rows_b, tr_b, swb)
        return carry

    lax.fori_loop(0, SEQ // 2, body, 0)
    drain_writes(0, tr_a, swa)
    drain_writes(0, tr_b, swb)


def kernel(input_data, table):
    idx = input_data.T.reshape(TOTAL).astype(jnp.int32)
    out = _lookup_kernel(idx, table)
    return jnp.transpose(out, (2, 0, 1))
